# Initial kernel scaffold; baseline (speedup 1.0000x reference)
import jax
import jax.numpy as jnp
from jax.experimental import pallas as pl


def kernel(x, edge_index, W1, b1, W2, b2):
    raise NotImplementedError("write your pallas kernel here")


# speedup vs baseline: 2.5972x; 2.5972x over previous
import functools

import jax
import jax.numpy as jnp
from jax import lax
from jax.experimental import pallas as pl
from jax.experimental.pallas import tpu as pltpu
import jax.experimental.pallas.tpu_sc as plsc

N = 50000
E = 800000
IN_DIM = 64
HID_DIM = 128
OUT_DIM = 128

NCORE = 2
NSUB = 16
NTILE = NCORE * NSUB

EP = 819200
CH = 13056
NPASS = 2
D_PAD = CH * NCORE * NPASS
CH_ROWS = CH + 16
ZROWS = CH_ROWS // NSUB
CP_ROWS = CH // NSUB
G = 128
DG = 512

DEG_SLICE = EP // NTILE
ACC_SLICE = EP // NSUB
DEG_PAD = 51200
DEG_ZCH = DEG_PAD // NSUB


def _mesh():
    return plsc.VectorSubcoreMesh(core_axis_name="c", subcore_axis_name="s",
                                  num_cores=NCORE, num_subcores=NSUB)


@functools.partial(
    pl.kernel,
    out_type=jax.ShapeDtypeStruct((NCORE * DEG_PAD,), jnp.float32),
    mesh=_mesh(),
    scratch_types=[
        pltpu.VMEM_SHARED((DEG_PAD,), jnp.float32),
        pltpu.VMEM((DG,), jnp.int32),
        pltpu.VMEM((DG,), jnp.float32),
        pltpu.VMEM((DEG_ZCH,), jnp.float32),
        pltpu.SemaphoreType.DMA,
    ],
    compiler_params=pltpu.CompilerParams(use_tc_tiling_on_sc=False),
)
def _deg_kernel(dst_hbm, ones_hbm, zeros1_hbm, out_hbm,
                acc_sh, dbuf, ones_v, stage, sem):
    c = lax.axis_index("c")
    s = lax.axis_index("s")
    pltpu.sync_copy(ones_hbm, ones_v)
    pltpu.sync_copy(zeros1_hbm, stage)
    pltpu.sync_copy(stage, acc_sh.at[pl.ds(s * DEG_ZCH, DEG_ZCH)])
    plsc.subcore_barrier()

    base = (c * NSUB + s) * DEG_SLICE

    def body(grp, _):
        off = base + grp * DG
        pltpu.sync_copy(dst_hbm.at[pl.ds(off, DG)], dbuf)
        pltpu.sync_copy(ones_v, acc_sh.at[dbuf], add=True)
        return 0

    lax.fori_loop(0, DEG_SLICE // DG, body, 0)
    plsc.subcore_barrier()
    pltpu.sync_copy(acc_sh.at[pl.ds(s * DEG_ZCH, DEG_ZCH)], stage)
    pltpu.sync_copy(stage, out_hbm.at[pl.ds(c * DEG_PAD + s * DEG_ZCH, DEG_ZCH)])


@functools.partial(
    pl.kernel,
    out_type=jax.ShapeDtypeStruct((D_PAD, HID_DIM), jnp.float32),
    mesh=_mesh(),
    scratch_types=[
        pltpu.VMEM_SHARED((CH_ROWS, HID_DIM), jnp.float32),
        pltpu.VMEM((G,), jnp.int32),
        pltpu.VMEM((G,), jnp.int32),
        pltpu.VMEM((G,), jnp.int32),
        pltpu.VMEM((G, HID_DIM), jnp.float32),
        pltpu.VMEM((64, HID_DIM), jnp.float32),
        pltpu.SemaphoreType.DMA,
    ],
    compiler_params=pltpu.CompilerParams(use_tc_tiling_on_sc=False),
)
def _acc_kernel(g_hbm, src_hbm, dst_hbm, zeros2_hbm, out_hbm,
                acc_sh, sbuf, dbuf, lidx, rows, zbuf, sem):
    c = lax.axis_index("c")
    s = lax.axis_index("s")
    edge_base = s * ACC_SLICE
    pltpu.sync_copy(zeros2_hbm, zbuf)

    for p in range(NPASS):
        chunk = p * NCORE + c
        row_base = chunk * CH

        zoff = s * ZROWS
        for k in range(ZROWS // 64):
            pltpu.sync_copy(zbuf, acc_sh.at[pl.ds(zoff + k * 64, 64)])
        rem = ZROWS % 64
        if rem:
            pltpu.sync_copy(zbuf.at[pl.ds(0, rem)],
                            acc_sh.at[pl.ds(zoff + (ZROWS // 64) * 64, rem)])
        plsc.subcore_barrier()

        def body(grp, _):
            off = edge_base + grp * G
            pltpu.sync_copy(src_hbm.at[pl.ds(off, G)], sbuf)
            pltpu.sync_copy(dst_hbm.at[pl.ds(off, G)], dbuf)
            pltpu.async_copy(g_hbm.at[sbuf], rows, sem).wait()

            def lbody(i, _):
                dv = dbuf[pl.ds(i * 16, 16)]
                lv = dv - row_base
                ok = (lv >= 0) & (lv < CH)
                lidx[pl.ds(i * 16, 16)] = jnp.where(ok, lv, CH)
                return 0

            lax.fori_loop(0, G // 16, lbody, 0)
            pltpu.sync_copy(rows, acc_sh.at[lidx], add=True)
            return 0

        lax.fori_loop(0, ACC_SLICE // G, body, 0)
        plsc.subcore_barrier()
        coff = s * CP_ROWS
        done = 0
        for sz in [G] * (CP_ROWS // G) + ([CP_ROWS % G] if CP_ROWS % G else []):
            pltpu.sync_copy(acc_sh.at[pl.ds(coff + done, sz)],
                            rows.at[pl.ds(0, sz)])
            pltpu.sync_copy(rows.at[pl.ds(0, sz)],
                            out_hbm.at[pl.ds(row_base + coff + done, sz)])
            done += sz
        plsc.subcore_barrier()


RB = 400


def _dis(d0, d1):
    return lax.rsqrt(d0 + d1 + 1.0)


def _t1_body(x_ref, w_ref, d0_ref, d1_ref, o_ref):
    dis = _dis(d0_ref[...], d1_ref[...])
    h = jnp.dot(x_ref[...], w_ref[...], preferred_element_type=jnp.float32)
    o_ref[...] = h * dis


def _t2_body(acc_ref, g_ref, d0_ref, d1_ref, b_ref, w_ref, o_ref):
    dis = _dis(d0_ref[...], d1_ref[...])
    z = jnp.maximum(dis * (acc_ref[...] + g_ref[...]) + b_ref[...], 0.0)
    o_ref[...] = jnp.dot(z, w_ref[...], preferred_element_type=jnp.float32) * dis


def _t3_body(acc_ref, g_ref, d0_ref, d1_ref, b_ref, o_ref):
    dis = _dis(d0_ref[...], d1_ref[...])
    o_ref[...] = dis * (acc_ref[...] + g_ref[...]) + b_ref[...]


def _row_spec(cols):
    return pl.BlockSpec((RB, cols), lambda i: (i, 0))


def _full_spec(r, c):
    return pl.BlockSpec((r, c), lambda i: (0, 0))


def _t1(x, w, d0, d1):
    return pl.pallas_call(
        _t1_body,
        grid=(N // RB,),
        in_specs=[_row_spec(IN_DIM), _full_spec(IN_DIM, HID_DIM),
                  _row_spec(1), _row_spec(1)],
        out_specs=_row_spec(HID_DIM),
        out_shape=jax.ShapeDtypeStruct((N, HID_DIM), jnp.float32),
    )(x, w, d0, d1)


def _t2(acc, g, d0, d1, b, w):
    return pl.pallas_call(
        _t2_body,
        grid=(N // RB,),
        in_specs=[_row_spec(HID_DIM), _row_spec(HID_DIM), _row_spec(1),
                  _row_spec(1), _full_spec(1, HID_DIM),
                  _full_spec(HID_DIM, OUT_DIM)],
        out_specs=_row_spec(OUT_DIM),
        out_shape=jax.ShapeDtypeStruct((N, OUT_DIM), jnp.float32),
    )(acc, g, d0, d1, b, w)


def _t3(acc, g, d0, d1, b):
    return pl.pallas_call(
        _t3_body,
        grid=(N // RB,),
        in_specs=[_row_spec(OUT_DIM), _row_spec(OUT_DIM), _row_spec(1),
                  _row_spec(1), _full_spec(1, OUT_DIM)],
        out_specs=_row_spec(OUT_DIM),
        out_shape=jax.ShapeDtypeStruct((N, OUT_DIM), jnp.float32),
    )(acc, g, d0, d1, b)


def kernel(x, edge_index, W1, b1, W2, b2):
    pad = EP - E
    src = jnp.concatenate([edge_index[0], jnp.zeros((pad,), jnp.int32)])
    dst = jnp.concatenate([edge_index[1], jnp.full((pad,), N, jnp.int32)])

    ones_g = jnp.ones((DG,), jnp.float32)
    zeros1 = jnp.zeros((DEG_ZCH,), jnp.float32)
    zeros2 = jnp.zeros((64, HID_DIM), jnp.float32)

    deg2 = _deg_kernel(dst, ones_g, zeros1)
    d0 = deg2[:N].reshape(N, 1)
    d1 = deg2[DEG_PAD:DEG_PAD + N].reshape(N, 1)

    g1 = _t1(x, W1, d0, d1)
    acc1 = _acc_kernel(g1, src, dst, zeros2)[:N]
    g2 = _t2(acc1, g1, d0, d1, b1.reshape(1, HID_DIM), W2)
    acc2 = _acc_kernel(g2, src, dst, zeros2)[:N]
    return _t3(acc2, g2, d0, d1, b2.reshape(1, OUT_DIM))
